# Initial kernel scaffold; baseline (speedup 1.0000x reference)
import jax
import jax.numpy as jnp
from jax.experimental import pallas as pl


def kernel(x):
    raise NotImplementedError("write your pallas kernel here")


# speedup vs baseline: 6.2226x; 6.2226x over previous
import functools

import jax
import jax.numpy as jnp
import numpy as np
from jax import lax
from jax.experimental import pallas as pl
from jax.experimental.pallas import tpu as pltpu
from jax.experimental.pallas import tpu_sc as plsc

N = 16777216
NC = 2
NS = 16
L = 16
NW = NC * NS
PER_W = N // NW
CH = 16384
NCH = PER_W // CH
UNROLL = 4
INNER = CH // (L * UNROLL)

_C_MAGIC = 12582912.0
_C_INV57344 = float(np.float32(1.0) / np.float32(57344.0))
_C_INV7 = float(np.float32(1.0) / np.float32(7.0))
_INV16384 = float(np.float32(1.0) / np.float32(16384.0))

_mesh = plsc.VectorSubcoreMesh(core_axis_name="c", subcore_axis_name="s")


def _make_tables():
    x_pts_fp = jnp.linspace(-10.0, 4.0, 17)
    ev = jnp.round(jnp.exp(x_pts_fp) * 16384.0).astype(jnp.int32)
    y0t = jnp.concatenate([ev, jnp.zeros((15,), jnp.int32)])
    dyt = jnp.concatenate([ev[1:] - ev[:-1], jnp.zeros((16,), jnp.int32)])
    return y0t, dyt


def _compute_chunk(xref, oref, y0t, dyt):
    def body(i, carry):
        for jj in range(UNROLL):
            off = i * (L * UNROLL) + jj * L
            xv = xref[pl.ds(off, L)]
            y = xv * 65536.0
            r = (y + _C_MAGIC) - _C_MAGIC
            w = jnp.maximum(r + 655359.0, -1.0)
            idxf = jnp.minimum(
                jnp.maximum(w * _C_INV57344, w - 917487.0), 16.0)
            idx = idxf.astype(jnp.int32)
            idxff = idx.astype(jnp.float32)
            nf = (w - idxff * 57344.0) * 2.0 + 5.0
            t = (nf * _C_INV7).astype(jnp.int32)
            y0 = plsc.load_gather(y0t, [idx])
            dy = plsc.load_gather(dyt, [idx])
            oi = y0 + ((t * dy + 8192) >> 14)
            oref[pl.ds(off, L)] = oi.astype(jnp.float32) * _INV16384
        return carry

    lax.fori_loop(0, INNER, body, 0)


@functools.partial(
    pl.kernel,
    mesh=_mesh,
    compiler_params=pltpu.CompilerParams(needs_layout_passes=False),
    out_type=jax.ShapeDtypeStruct((N,), jnp.float32),
    scratch_types=[
        pltpu.VMEM((CH,), jnp.float32),
        pltpu.VMEM((CH,), jnp.float32),
        pltpu.VMEM((CH,), jnp.float32),
        pltpu.VMEM((CH,), jnp.float32),
        pltpu.VMEM((32,), jnp.int32),
        pltpu.VMEM((32,), jnp.int32),
        pltpu.SemaphoreType.DMA,
        pltpu.SemaphoreType.DMA,
        pltpu.SemaphoreType.DMA,
        pltpu.SemaphoreType.DMA,
    ],
)
def _sc_exp_kernel(x_hbm, y0_hbm, dy_hbm, out_hbm,
                   xb0, xb1, ob0, ob1, y0t, dyt, si0, si1, so0, so1):
    wid = lax.axis_index("s") * NC + lax.axis_index("c")
    base = wid * PER_W

    pltpu.sync_copy(y0_hbm, y0t)
    pltpu.sync_copy(dy_hbm, dyt)

    xbs = (xb0, xb1)
    obs = (ob0, ob1)
    sis = (si0, si1)
    sos = (so0, so1)

    in_h = [
        pltpu.async_copy(x_hbm.at[pl.ds(base + b * CH, CH)], xbs[b], sis[b])
        for b in range(2)
    ]
    out_h = [None, None]
    for g in range(NCH):
        b = g % 2
        in_h[b].wait()
        if out_h[b] is not None:
            out_h[b].wait()
        _compute_chunk(xbs[b], obs[b], y0t, dyt)
        if g + 2 < NCH:
            in_h[b] = pltpu.async_copy(
                x_hbm.at[pl.ds(base + (g + 2) * CH, CH)], xbs[b], sis[b])
        out_h[b] = pltpu.async_copy(
            obs[b], out_hbm.at[pl.ds(base + g * CH, CH)], sos[b])
    out_h[0].wait()
    out_h[1].wait()


def kernel(x):
    y0t, dyt = _make_tables()
    return _sc_exp_kernel(x, y0t, dyt)
